# baseline (speedup 1.0000x reference)
import functools

import jax
import jax.numpy as jnp
from jax import lax
from jax.experimental import pallas as pl
from jax.experimental.pallas import tpu as pltpu
from jax.experimental.pallas import tpu_sc as plsc

LANES = 16
NBUF = 3
CH = 8


def _make_sc_kernel(B, S, D):
    info = plsc.get_sparse_core_info()
    NC, NS = info.num_cores, info.num_subcores
    NW = NC * NS
    s_per_w = S // NW
    n_ch = s_per_w // CH
    n_col = D // LANES

    mesh = plsc.VectorSubcoreMesh(core_axis_name="c", subcore_axis_name="s")

    scratch = (
        [pltpu.VMEM((B, CH, D), jnp.float32) for _ in range(NBUF)]
        + [pltpu.VMEM((CH, D), jnp.float32) for _ in range(NBUF)]
        + [pltpu.SemaphoreType.DMA for _ in range(2 * NBUF)]
    )

    @functools.partial(
        pl.kernel,
        mesh=mesh,
        out_type=jax.ShapeDtypeStruct((B, S, D), jnp.float32),
        scratch_types=scratch,
        compiler_params=pltpu.CompilerParams(use_tc_tiling_on_sc=True),
    )
    def k(xf, pe, out, *refs):
        xbs = refs[:NBUF]
        pbs = refs[NBUF:2 * NBUF]
        lss = refs[2 * NBUF:3 * NBUF]
        sss = refs[3 * NBUF:4 * NBUF]

        wid = lax.axis_index("s") * NC + lax.axis_index("c")
        s_base = wid * s_per_w

        def start_loads(c):
            p = c % NBUF
            s0 = s_base + c * CH
            return [
                pltpu.async_copy(pe.at[pl.ds(s0, CH), :], pbs[p], lss[p]),
                pltpu.async_copy(xf.at[:, pl.ds(s0, CH), :], xbs[p], lss[p]),
            ]

        def start_stores(c):
            p = c % NBUF
            s0 = s_base + c * CH
            return [
                pltpu.async_copy(xbs[p], out.at[:, pl.ds(s0, CH), :], sss[p]),
            ]

        GPB = 4

        def compute(c):
            p = c % NBUF
            xb, pb = xbs[p], pbs[p]

            def body(r, carry):
                def cbody(j, carry2):
                    base = j * (GPB * LANES)
                    for g in range(GPB):
                        col = base + g * LANES
                        vec = pb[r, pl.ds(col, LANES)]
                        for b in range(B):
                            plsc.addupdate(xb.at[b, r, pl.ds(col, LANES)], vec)
                    return carry2

                lax.fori_loop(0, n_col // GPB, cbody, 0)
                return carry

            lax.fori_loop(0, CH, body, 0)

        loads = {c: start_loads(c) for c in range(min(NBUF, n_ch))}
        stores = {}
        for c in range(n_ch):
            if c >= NBUF - 1:
                for h in stores.pop(c - (NBUF - 1)):
                    h.wait()
                if c + 1 < n_ch:
                    loads[c + 1] = start_loads(c + 1)
            for h in loads.pop(c):
                h.wait()
            compute(c)
            stores[c] = start_stores(c)
        for hs in stores.values():
            for h in hs:
                h.wait()

    return k


def kernel(x, pe_weight):
    B, S, D = x.shape
    return _make_sc_kernel(B, S, D)(x, pe_weight[:S])
